# Initial kernel scaffold; baseline (speedup 1.0000x reference)
import functools

import jax
import jax.numpy as jnp
from jax import lax
from jax.experimental import pallas as pl
from jax.experimental.pallas import tpu as pltpu
from jax.experimental.pallas import tpu_sc as plsc

B, S, D = 4096, 50, 32
QS, US, VS, CS = 100000, 1000000, 100, 2
SMALL = VS + CS + S
BS = B * S


def _proj_body(x_ref, w_ref, o_ref):
    o_ref[...] = jax.lax.dot_general(
        x_ref[...], w_ref[...], (((1,), (1,)), ((), ())),
        preferred_element_type=jnp.float32)


def _project(table, w_row, blk):
    n = table.shape[0]
    grid = (n + blk - 1) // blk
    out = pl.pallas_call(
        _proj_body,
        grid=(grid,),
        in_specs=[
            pl.BlockSpec((blk, D), lambda i: (i, 0)),
            pl.BlockSpec((1, D), lambda i: (0, 0)),
        ],
        out_specs=pl.BlockSpec((blk, 1), lambda i: (i, 0)),
        out_shape=jax.ShapeDtypeStruct((n, 1), jnp.float32),
    )(table, w_row)
    return out[:, 0]


def _small_body(x_ref, we_ref, wr_ref, be_ref, br_ref, oe_ref, or_ref):
    x = x_ref[...]
    oe_ref[...] = jnp.sum(x * we_ref[...], axis=1, keepdims=True) + be_ref[...]
    or_ref[...] = jnp.sum(x * wr_ref[...], axis=1, keepdims=True) + br_ref[...]


def _project_small(small_table, w_e_mat, w_r_mat, bias_e, bias_r):
    oe, orr = pl.pallas_call(
        _small_body,
        out_shape=(
            jax.ShapeDtypeStruct((SMALL, 1), jnp.float32),
            jax.ShapeDtypeStruct((SMALL, 1), jnp.float32),
        ),
    )(small_table, w_e_mat, w_r_mat, bias_e, bias_r)
    return oe[:, 0], orr[:, 0]


def _make_sc_combine():
    info = plsc.get_sparse_core_info()
    nc, ns = info.num_cores, info.num_subcores
    nw = nc * ns
    chunk = BS // nw
    groups = chunk // 16

    mesh = plsc.VectorSubcoreMesh(core_axis_name="c", subcore_axis_name="s")

    @functools.partial(
        pl.kernel,
        mesh=mesh,
        out_type=(
            jax.ShapeDtypeStruct((BS,), jnp.float32),
            jax.ShapeDtypeStruct((BS,), jnp.float32),
        ),
        scratch_types=[
            pltpu.VMEM((chunk,), jnp.int32),
            pltpu.VMEM((chunk,), jnp.int32),
            pltpu.VMEM((chunk,), jnp.int32),
            pltpu.VMEM((chunk,), jnp.int32),
            pltpu.VMEM((chunk,), jnp.float32),
            pltpu.VMEM((chunk,), jnp.float32),
            pltpu.VMEM((SMALL,), jnp.float32),
            pltpu.VMEM((SMALL,), jnp.float32),
            pltpu.VMEM((chunk,), jnp.float32),
            pltpu.VMEM((chunk,), jnp.float32),
            pltpu.SemaphoreType.DMA,
            pltpu.SemaphoreType.DMA,
        ],
    )
    def sc_combine(qids_hbm, uids_hbm, vids_hbm, clicks_hbm,
                   q_proj_hbm, u_proj_hbm, small_e_hbm, small_r_hbm,
                   pred_hbm, rels_hbm,
                   qi_v, ui_v, vi_v, ci_v, qv_v, uv_v, se_v, sr_v,
                   op_v, or_v, sem0, sem1):
        wid = lax.axis_index("s") * nc + lax.axis_index("c")
        base = wid * chunk
        sl_all = pl.ds(base, chunk)
        pltpu.sync_copy(qids_hbm.at[sl_all], qi_v)
        pltpu.sync_copy(uids_hbm.at[sl_all], ui_v)
        pltpu.sync_copy(vids_hbm.at[sl_all], vi_v)
        pltpu.sync_copy(clicks_hbm.at[sl_all], ci_v)
        pltpu.sync_copy(small_e_hbm, se_v)
        pltpu.sync_copy(small_r_hbm, sr_v)
        cp_q = pltpu.async_copy(q_proj_hbm.at[qi_v], qv_v, sem0)
        cp_u = pltpu.async_copy(u_proj_hbm.at[ui_v], uv_v, sem1)
        cp_q.wait()
        cp_u.wait()

        lane = lax.iota(jnp.int32, 16)

        def body(g, carry):
            sl = pl.ds(g * 16, 16)
            vi = vi_v[sl]
            ci = ci_v[sl] + VS
            pi = lax.rem(base + g * 16 + lane, S) + (VS + CS)
            ve = plsc.load_gather(se_v, [vi])
            ce = plsc.load_gather(se_v, [ci])
            pe = plsc.load_gather(se_v, [pi])
            vr = plsc.load_gather(sr_v, [vi])
            cr = plsc.load_gather(sr_v, [ci])
            pr = plsc.load_gather(sr_v, [pi])
            ex = 1.0 / (1.0 + jnp.exp(-(ve + ce + pe)))
            rl = 1.0 / (1.0 + jnp.exp(-(qv_v[sl] + uv_v[sl] + vr + cr + pr)))
            or_v[sl] = rl
            op_v[sl] = rl * ex
            return carry

        lax.fori_loop(0, groups, body, 0)

        pltpu.sync_copy(op_v, pred_hbm.at[sl_all])
        pltpu.sync_copy(or_v, rels_hbm.at[sl_all])

    return sc_combine


_sc_combine = None


def kernel(qids, uids, vids, clicks, q_table, u_table, v_table, click_table,
           pos_table, w_exam, b_exam, w_rel, b_rel):
    global _sc_combine
    if _sc_combine is None:
        _sc_combine = _make_sc_combine()

    u_proj = _project(u_table, w_rel[D:2 * D].reshape(1, D), 16384)
    q_proj = _project(q_table, w_rel[:D].reshape(1, D), 12544)

    small_table = jnp.concatenate([v_table, click_table, pos_table], axis=0)
    w_e_mat = jnp.concatenate([
        jnp.broadcast_to(w_exam[:D], (VS, D)),
        jnp.broadcast_to(w_exam[D:2 * D], (CS, D)),
        jnp.broadcast_to(w_exam[2 * D:], (S, D)),
    ], axis=0)
    w_r_mat = jnp.concatenate([
        jnp.broadcast_to(w_rel[2 * D:3 * D], (VS, D)),
        jnp.broadcast_to(w_rel[3 * D:4 * D], (CS, D)),
        jnp.broadcast_to(w_rel[4 * D:], (S, D)),
    ], axis=0)
    rows = jnp.arange(SMALL)
    click_row = ((rows >= VS) & (rows < VS + CS)).astype(jnp.float32)
    bias_e = (click_row * b_exam[0]).reshape(SMALL, 1)
    bias_r = (click_row * b_rel[0]).reshape(SMALL, 1)
    small_e, small_r = _project_small(small_table, w_e_mat, w_r_mat,
                                      bias_e, bias_r)

    pred_f, rels_f = _sc_combine(
        qids.reshape(BS), uids.reshape(BS), vids.reshape(BS),
        clicks.reshape(BS), q_proj, u_proj, small_e, small_r)
    return pred_f.reshape(B, S), rels_f.reshape(B, S)


# speedup vs baseline: 4.2342x; 4.2342x over previous
import functools

import jax
import jax.numpy as jnp
from jax import lax
from jax.experimental import pallas as pl
from jax.experimental.pallas import tpu as pltpu
from jax.experimental.pallas import tpu_sc as plsc

B, S, D = 4096, 50, 32
QS, US, VS, CS = 100000, 1000000, 100, 2
SMALL = VS + CS + S
SPAD = 256
BS = B * S


def _proj_body(x_ref, w_ref, o_ref):
    o_ref[...] = jax.lax.dot_general(
        x_ref[...], w_ref[...], (((1,), (1,)), ((), ())),
        preferred_element_type=jnp.float32)


def _project(table, w_row, blk):
    n = table.shape[0]
    grid = (n + blk - 1) // blk
    out = pl.pallas_call(
        _proj_body,
        grid=(grid,),
        in_specs=[
            pl.BlockSpec((blk, D), lambda i: (i, 0)),
            pl.BlockSpec((1, D), lambda i: (0, 0)),
        ],
        out_specs=pl.BlockSpec((blk, 1), lambda i: (i, 0)),
        out_shape=jax.ShapeDtypeStruct((n, 1), jnp.float32),
    )(table, w_row)
    return out[:, 0]


def _small_body(x_ref, we_ref, wr_ref, be_ref, br_ref, oe_ref, or_ref):
    x = x_ref[...]
    oe_ref[...] = jnp.sum(x * we_ref[...], axis=1, keepdims=True) + be_ref[...]
    or_ref[...] = jnp.sum(x * wr_ref[...], axis=1, keepdims=True) + br_ref[...]


def _project_small(small_table, w_e_mat, w_r_mat, bias_e, bias_r):
    oe, orr = pl.pallas_call(
        _small_body,
        out_shape=(
            jax.ShapeDtypeStruct((SPAD, 1), jnp.float32),
            jax.ShapeDtypeStruct((SPAD, 1), jnp.float32),
        ),
    )(small_table, w_e_mat, w_r_mat, bias_e, bias_r)
    return oe[:, 0], orr[:, 0]


def _make_sc_combine():
    info = plsc.get_sparse_core_info()
    nc, ns = info.num_cores, info.num_subcores
    nw = nc * ns
    chunk = BS // nw
    groups = chunk // 16

    mesh = plsc.VectorSubcoreMesh(core_axis_name="c", subcore_axis_name="s")

    @functools.partial(
        pl.kernel,
        mesh=mesh,
        out_type=(
            jax.ShapeDtypeStruct((BS,), jnp.float32),
            jax.ShapeDtypeStruct((BS,), jnp.float32),
        ),
        scratch_types=[
            pltpu.VMEM((chunk,), jnp.int32),
            pltpu.VMEM((chunk,), jnp.int32),
            pltpu.VMEM((chunk,), jnp.int32),
            pltpu.VMEM((chunk,), jnp.int32),
            pltpu.VMEM((chunk,), jnp.float32),
            pltpu.VMEM((chunk,), jnp.float32),
            pltpu.VMEM((chunk,), jnp.float32),
            pltpu.VMEM((chunk,), jnp.float32),
            pltpu.VMEM((chunk,), jnp.float32),
            pltpu.VMEM((chunk,), jnp.float32),
            pltpu.VMEM((16,), jnp.float32),
            pltpu.VMEM((16,), jnp.float32),
            pltpu.VMEM((chunk,), jnp.float32),
            pltpu.VMEM((chunk,), jnp.float32),
            pltpu.SemaphoreType.DMA,
            pltpu.SemaphoreType.DMA,
            pltpu.SemaphoreType.DMA,
            pltpu.SemaphoreType.DMA,
        ],
    )
    def sc_combine(qids_hbm, uids_hbm, vids_hbm, clicks_hbm,
                   q_proj_hbm, u_proj_hbm, ve_hbm, vr_hbm,
                   pe_hbm, pr_hbm, dce_hbm, dcr_hbm,
                   pred_hbm, rels_hbm,
                   qi_v, ui_v, vi_v, ci_v, qv_v, uv_v, vve_v, vvr_v,
                   pe_v, pr_v, dce_v, dcr_v, op_v, or_v,
                   sem0, sem1, sem2, sem3):
        wid = lax.axis_index("s") * nc + lax.axis_index("c")
        base = wid * chunk
        sl_all = pl.ds(base, chunk)
        pltpu.sync_copy(qids_hbm.at[sl_all], qi_v)
        pltpu.sync_copy(uids_hbm.at[sl_all], ui_v)
        pltpu.sync_copy(vids_hbm.at[sl_all], vi_v)
        pltpu.sync_copy(clicks_hbm.at[sl_all], ci_v)
        pltpu.sync_copy(pe_hbm, pe_v)
        pltpu.sync_copy(pr_hbm, pr_v)
        pltpu.sync_copy(dce_hbm, dce_v)
        pltpu.sync_copy(dcr_hbm, dcr_v)
        cp_q = pltpu.async_copy(q_proj_hbm.at[qi_v], qv_v, sem0)
        cp_u = pltpu.async_copy(u_proj_hbm.at[ui_v], uv_v, sem1)
        cp_ve = pltpu.async_copy(ve_hbm.at[vi_v], vve_v, sem2)
        cp_vr = pltpu.async_copy(vr_hbm.at[vi_v], vvr_v, sem3)
        cp_q.wait()
        cp_u.wait()
        cp_ve.wait()
        cp_vr.wait()

        dce = dce_v[...]
        dcr = dcr_v[...]

        def body(g, carry):
            sl = pl.ds(g * 16, 16)
            cf = ci_v[sl].astype(jnp.float32)
            ex_logit = vve_v[sl] + cf * dce + pe_v[sl]
            rl_logit = qv_v[sl] + uv_v[sl] + vvr_v[sl] + cf * dcr + pr_v[sl]
            ex = 1.0 / (1.0 + jnp.exp(-ex_logit))
            rl = 1.0 / (1.0 + jnp.exp(-rl_logit))
            or_v[sl] = rl
            op_v[sl] = rl * ex
            return carry

        lax.fori_loop(0, groups, body, 0)

        pltpu.sync_copy(op_v, pred_hbm.at[sl_all])
        pltpu.sync_copy(or_v, rels_hbm.at[sl_all])

    return sc_combine


_sc_combine = None


def kernel(qids, uids, vids, clicks, q_table, u_table, v_table, click_table,
           pos_table, w_exam, b_exam, w_rel, b_rel):
    global _sc_combine
    if _sc_combine is None:
        _sc_combine = _make_sc_combine()

    u_proj = _project(u_table, w_rel[D:2 * D].reshape(1, D), 16384)
    q_proj = _project(q_table, w_rel[:D].reshape(1, D), 12544)

    zpad = jnp.zeros((SPAD - SMALL, D), jnp.float32)
    small_table = jnp.concatenate([v_table, click_table, pos_table, zpad],
                                  axis=0)
    w_e_mat = jnp.concatenate([
        jnp.broadcast_to(w_exam[:D], (VS, D)),
        jnp.broadcast_to(w_exam[D:2 * D], (CS, D)),
        jnp.broadcast_to(w_exam[2 * D:], (S, D)),
        zpad,
    ], axis=0)
    w_r_mat = jnp.concatenate([
        jnp.broadcast_to(w_rel[2 * D:3 * D], (VS, D)),
        jnp.broadcast_to(w_rel[3 * D:4 * D], (CS, D)),
        jnp.broadcast_to(w_rel[4 * D:], (S, D)),
        zpad,
    ], axis=0)
    rows = jnp.arange(SPAD)
    click_row = ((rows >= VS) & (rows < VS + CS)).astype(jnp.float32)
    bias_e = (click_row * b_exam[0]).reshape(SPAD, 1)
    bias_r = (click_row * b_rel[0]).reshape(SPAD, 1)
    small_e, small_r = _project_small(small_table, w_e_mat, w_r_mat,
                                      bias_e, bias_r)

    v_e = small_e[:VS]
    v_r = small_r[:VS]
    ce0, ce1 = small_e[VS], small_e[VS + 1]
    cr0, cr1 = small_r[VS], small_r[VS + 1]
    p_e = small_e[VS + CS:SMALL]
    p_r = small_r[VS + CS:SMALL]
    pe_chunk = jnp.tile(p_e, 128) + ce0
    pr_chunk = jnp.tile(p_r, 128) + cr0
    dce16 = jnp.broadcast_to(ce1 - ce0, (16,))
    dcr16 = jnp.broadcast_to(cr1 - cr0, (16,))

    pred_f, rels_f = _sc_combine(
        qids.reshape(BS), uids.reshape(BS), vids.reshape(BS),
        clicks.reshape(BS), q_proj, u_proj, v_e, v_r,
        pe_chunk, pr_chunk, dce16, dcr16)
    return pred_f.reshape(B, S), rels_f.reshape(B, S)


# speedup vs baseline: 6.6885x; 1.5797x over previous
import functools

import jax
import jax.numpy as jnp
from jax import lax
from jax.experimental import pallas as pl
from jax.experimental.pallas import tpu as pltpu
from jax.experimental.pallas import tpu_sc as plsc

B, S, D = 4096, 50, 32
QS, US, VS, CS = 100000, 1000000, 100, 2
SMALL = VS + CS + S
SPAD = 256
BS = B * S
UBLK, QBLK = 15872, 12544
USP = ((US + UBLK - 1) // UBLK) * UBLK
QSP = ((QS + QBLK - 1) // QBLK) * QBLK
UPS = USP // 2
NCH_F = 10


def _proj_body_f32(x_ref, w_ref, o_ref):
    o_ref[...] = jax.lax.dot_general(
        x_ref[...], w_ref[...], (((1,), (1,)), ((), ())),
        preferred_element_type=jnp.float32)


def _proj_body_bf16(x_ref, w_ref, o_ref):
    o_ref[...] = jax.lax.dot_general(
        x_ref[...], w_ref[...], (((1,), (1,)), ((), ())),
        preferred_element_type=jnp.float32).astype(jnp.bfloat16)


def _project(table, w_row, blk, dtype=jnp.float32):
    n = table.shape[0]
    grid = (n + blk - 1) // blk
    body = _proj_body_f32 if dtype == jnp.float32 else _proj_body_bf16
    out = pl.pallas_call(
        body,
        grid=(grid,),
        in_specs=[
            pl.BlockSpec((blk, D), lambda i: (i, 0)),
            pl.BlockSpec((1, D), lambda i: (0, 0)),
        ],
        out_specs=pl.BlockSpec((blk, 1), lambda i: (i, 0)),
        out_shape=jax.ShapeDtypeStruct((grid * blk, 1), dtype),
    )(table, w_row)
    return out[:, 0]


def _small_body(x_ref, we_ref, wr_ref, be_ref, br_ref, oe_ref, or_ref):
    x = x_ref[...]
    oe_ref[...] = jnp.sum(x * we_ref[...], axis=1, keepdims=True) + be_ref[...]
    or_ref[...] = jnp.sum(x * wr_ref[...], axis=1, keepdims=True) + br_ref[...]


def _project_small(small_table, w_e_mat, w_r_mat, bias_e, bias_r):
    oe, orr = pl.pallas_call(
        _small_body,
        out_shape=(
            jax.ShapeDtypeStruct((SPAD, 1), jnp.float32),
            jax.ShapeDtypeStruct((SPAD, 1), jnp.float32),
        ),
    )(small_table, w_e_mat, w_r_mat, bias_e, bias_r)
    return oe[:, 0], orr[:, 0]


def _make_sc_combine():
    info = plsc.get_sparse_core_info()
    nc, ns = info.num_cores, info.num_subcores
    nw = nc * ns
    chunk = BS // nw
    groups = chunk // 16

    mesh = plsc.VectorSubcoreMesh(core_axis_name="c", subcore_axis_name="s")

    @functools.partial(
        pl.kernel,
        mesh=mesh,
        out_type=(
            jax.ShapeDtypeStruct((BS,), jnp.float32),
            jax.ShapeDtypeStruct((BS,), jnp.float32),
        ),
        scratch_types=[
            pltpu.VMEM((chunk,), jnp.int32),
            pltpu.VMEM((chunk,), jnp.int32),
            pltpu.VMEM((chunk,), jnp.int32),
            pltpu.VMEM((chunk,), jnp.int32),
            pltpu.VMEM((chunk,), jnp.int32),
            pltpu.VMEM((chunk,), jnp.float32),
            pltpu.VMEM((chunk,), jnp.int32),
            pltpu.VMEM((chunk,), jnp.float32),
            pltpu.VMEM((chunk,), jnp.float32),
            pltpu.VMEM((chunk,), jnp.float32),
            pltpu.VMEM((chunk,), jnp.float32),
            pltpu.VMEM((16,), jnp.float32),
            pltpu.VMEM((16,), jnp.float32),
            pltpu.VMEM((chunk,), jnp.float32),
            pltpu.VMEM((chunk,), jnp.float32),
            pltpu.VMEM_SHARED((QSP,), jnp.float32),
            pltpu.VMEM_SHARED((UPS,), jnp.int32),
            pltpu.VMEM_SHARED((128,), jnp.float32),
            pltpu.VMEM_SHARED((128,), jnp.float32),
            pltpu.SemaphoreType.DMA,
            pltpu.SemaphoreType.DMA,
            pltpu.SemaphoreType.DMA,
            pltpu.SemaphoreType.DMA,
        ],
    )
    def sc_combine(qids_hbm, uids_hbm, upair_hbm, vids_hbm, clicks_hbm,
                   q_proj_hbm, u_pairs_hbm, ve_hbm, vr_hbm,
                   pe_hbm, pr_hbm, dce_hbm, dcr_hbm,
                   pred_hbm, rels_hbm,
                   qi_v, ui_v, upi_v, vi_v, ci_v, qv_v, uv_v, vve_v, vvr_v,
                   pe_v, pr_v, dce_v, dcr_v, op_v, or_v,
                   q_sh, u_sh, ve_sh, vr_sh,
                   sem0, sem1, sem2, sem3):
        sid = lax.axis_index("s")
        wid = sid * nc + lax.axis_index("c")
        base = wid * chunk
        sl_all = pl.ds(base, chunk)
        upart = UPS // ns
        u0 = sid * upart
        pltpu.sync_copy(u_pairs_hbm.at[pl.ds(u0, upart)],
                        u_sh.at[pl.ds(u0, upart)])
        qpart = QSP // ns
        q0 = sid * qpart
        pltpu.sync_copy(q_proj_hbm.at[pl.ds(q0, qpart)],
                        q_sh.at[pl.ds(q0, qpart)])

        @pl.when(sid == 0)
        def _stage_small():
            pltpu.sync_copy(ve_hbm, ve_sh)
            pltpu.sync_copy(vr_hbm, vr_sh)

        pltpu.sync_copy(qids_hbm.at[sl_all], qi_v)
        pltpu.sync_copy(uids_hbm.at[sl_all], ui_v)
        pltpu.sync_copy(upair_hbm.at[sl_all], upi_v)
        pltpu.sync_copy(vids_hbm.at[sl_all], vi_v)
        pltpu.sync_copy(clicks_hbm.at[sl_all], ci_v)
        pltpu.sync_copy(pe_hbm, pe_v)
        pltpu.sync_copy(pr_hbm, pr_v)
        pltpu.sync_copy(dce_hbm, dce_v)
        pltpu.sync_copy(dcr_hbm, dcr_v)
        plsc.subcore_barrier()
        copies = []
        cf_ = chunk // NCH_F
        for c in range(NCH_F):
            sl = pl.ds(c * cf_, cf_)
            copies.append(pltpu.async_copy(u_sh.at[upi_v.at[sl]],
                                           uv_v.at[sl], sem1))
            copies.append(pltpu.async_copy(q_sh.at[qi_v.at[sl]],
                                           qv_v.at[sl], sem0))
            copies.append(pltpu.async_copy(ve_sh.at[vi_v.at[sl]],
                                           vve_v.at[sl], sem2))
            copies.append(pltpu.async_copy(vr_sh.at[vi_v.at[sl]],
                                           vvr_v.at[sl], sem3))
        for cp in copies:
            cp.wait()

        dce = dce_v[...]
        dcr = dcr_v[...]
        one = jnp.full((16,), 1, jnp.int32)
        himask = jnp.full((16,), -65536, jnp.int32)
        sixteen = jnp.full((16,), 16, jnp.int32)

        def body(g, carry):
            sl = pl.ds(g * 16, 16)
            up = uv_v[sl]
            lsb = lax.bitwise_and(ui_v[sl], one)
            sham = lax.mul(lax.sub(one, lsb), sixteen)
            ubits = lax.bitwise_and(lax.shift_left(up, sham), himask)
            uval = lax.bitcast_convert_type(ubits, jnp.float32)
            cf = ci_v[sl].astype(jnp.float32)
            ex_logit = vve_v[sl] + cf * dce + pe_v[sl]
            rl_logit = qv_v[sl] + uval + vvr_v[sl] + cf * dcr + pr_v[sl]
            ex = 1.0 / (1.0 + jnp.exp(-ex_logit))
            rl = 1.0 / (1.0 + jnp.exp(-rl_logit))
            or_v[sl] = rl
            op_v[sl] = rl * ex
            return carry

        lax.fori_loop(0, groups, body, 0)

        pltpu.sync_copy(op_v, pred_hbm.at[sl_all])
        pltpu.sync_copy(or_v, rels_hbm.at[sl_all])

    return sc_combine


_sc_combine = None


def kernel(qids, uids, vids, clicks, q_table, u_table, v_table, click_table,
           pos_table, w_exam, b_exam, w_rel, b_rel):
    global _sc_combine
    if _sc_combine is None:
        _sc_combine = _make_sc_combine()

    u_proj = _project(u_table, w_rel[D:2 * D].reshape(1, D), UBLK,
                      jnp.bfloat16)
    q_proj = _project(q_table, w_rel[:D].reshape(1, D), QBLK)

    zpad = jnp.zeros((SPAD - SMALL, D), jnp.float32)
    small_table = jnp.concatenate([v_table, click_table, pos_table, zpad],
                                  axis=0)
    w_e_mat = jnp.concatenate([
        jnp.broadcast_to(w_exam[:D], (VS, D)),
        jnp.broadcast_to(w_exam[D:2 * D], (CS, D)),
        jnp.broadcast_to(w_exam[2 * D:], (S, D)),
        zpad,
    ], axis=0)
    w_r_mat = jnp.concatenate([
        jnp.broadcast_to(w_rel[2 * D:3 * D], (VS, D)),
        jnp.broadcast_to(w_rel[3 * D:4 * D], (CS, D)),
        jnp.broadcast_to(w_rel[4 * D:], (S, D)),
        zpad,
    ], axis=0)
    rows = jnp.arange(SPAD)
    click_row = ((rows >= VS) & (rows < VS + CS)).astype(jnp.float32)
    bias_e = (click_row * b_exam[0]).reshape(SPAD, 1)
    bias_r = (click_row * b_rel[0]).reshape(SPAD, 1)
    small_e, small_r = _project_small(small_table, w_e_mat, w_r_mat,
                                      bias_e, bias_r)

    v_e = jnp.concatenate([small_e[:VS], jnp.zeros((128 - VS,), jnp.float32)])
    v_r = jnp.concatenate([small_r[:VS], jnp.zeros((128 - VS,), jnp.float32)])
    ce0, ce1 = small_e[VS], small_e[VS + 1]
    cr0, cr1 = small_r[VS], small_r[VS + 1]
    p_e = small_e[VS + CS:SMALL]
    p_r = small_r[VS + CS:SMALL]
    pe_chunk = jnp.tile(p_e, 128) + ce0
    pr_chunk = jnp.tile(p_r, 128) + cr0
    dce16 = jnp.broadcast_to(ce1 - ce0, (16,))
    dcr16 = jnp.broadcast_to(cr1 - cr0, (16,))

    u_pairs = lax.bitcast_convert_type(u_proj.reshape(UPS, 2), jnp.int32)
    uids_f = uids.reshape(BS)
    pred_f, rels_f = _sc_combine(
        qids.reshape(BS), uids_f, lax.shift_right_logical(uids_f, 1),
        vids.reshape(BS), clicks.reshape(BS), q_proj, u_pairs, v_e, v_r,
        pe_chunk, pr_chunk, dce16, dcr16)
    return pred_f.reshape(B, S), rels_f.reshape(B, S)


# speedup vs baseline: 16.2002x; 2.4221x over previous
import functools

import jax
import jax.numpy as jnp
from jax import lax
from jax.experimental import pallas as pl
from jax.experimental.pallas import tpu as pltpu
from jax.experimental.pallas import tpu_sc as plsc

B, S, D = 4096, 50, 32
QS, US, VS, CS = 100000, 1000000, 100, 2
SMALL = VS + CS + S
SPAD = 256
BS = B * S
UBLK, QBLK = 32768, 13312
USP = ((US + UBLK - 1) // UBLK) * UBLK
QSP = ((QS + QBLK - 1) // QBLK) * QBLK
UPS = USP // 2
NCH_F = 10


def _proj_body_f32(x_ref, w_ref, o_ref):
    o_ref[...] = jnp.sum(x_ref[...] * w_ref[...], axis=0)


def _proj_body_bf16(x_ref, w_ref, o_ref):
    o_ref[...] = jnp.sum(x_ref[...] * w_ref[...], axis=0).astype(jnp.bfloat16)


def _project(table_t, w_col, blk, dtype=jnp.float32):
    n = table_t.shape[1]
    grid = (n + blk - 1) // blk
    body = _proj_body_f32 if dtype == jnp.float32 else _proj_body_bf16
    out = pl.pallas_call(
        body,
        grid=(grid,),
        in_specs=[
            pl.BlockSpec((D, blk), lambda i: (0, i)),
            pl.BlockSpec((D, 1), lambda i: (0, 0)),
        ],
        out_specs=pl.BlockSpec((blk,), lambda i: (i,)),
        out_shape=jax.ShapeDtypeStruct((grid * blk,), dtype),
    )(table_t, w_col)
    return out


def _small_body(x_ref, we_ref, wr_ref, be_ref, br_ref, oe_ref, or_ref):
    x = x_ref[...]
    oe_ref[...] = jnp.sum(x * we_ref[...], axis=1, keepdims=True) + be_ref[...]
    or_ref[...] = jnp.sum(x * wr_ref[...], axis=1, keepdims=True) + br_ref[...]


def _project_small(small_table, w_e_mat, w_r_mat, bias_e, bias_r):
    oe, orr = pl.pallas_call(
        _small_body,
        out_shape=(
            jax.ShapeDtypeStruct((SPAD, 1), jnp.float32),
            jax.ShapeDtypeStruct((SPAD, 1), jnp.float32),
        ),
    )(small_table, w_e_mat, w_r_mat, bias_e, bias_r)
    return oe[:, 0], orr[:, 0]


def _make_sc_combine():
    info = plsc.get_sparse_core_info()
    nc, ns = info.num_cores, info.num_subcores
    nw = nc * ns
    chunk = BS // nw
    groups = chunk // 16

    mesh = plsc.VectorSubcoreMesh(core_axis_name="c", subcore_axis_name="s")

    @functools.partial(
        pl.kernel,
        mesh=mesh,
        out_type=(
            jax.ShapeDtypeStruct((BS,), jnp.float32),
            jax.ShapeDtypeStruct((BS,), jnp.float32),
        ),
        scratch_types=[
            pltpu.VMEM((chunk,), jnp.int32),
            pltpu.VMEM((chunk,), jnp.int32),
            pltpu.VMEM((chunk,), jnp.int32),
            pltpu.VMEM((chunk,), jnp.int32),
            pltpu.VMEM((chunk,), jnp.int32),
            pltpu.VMEM((chunk,), jnp.float32),
            pltpu.VMEM((chunk,), jnp.int32),
            pltpu.VMEM((chunk,), jnp.float32),
            pltpu.VMEM((chunk,), jnp.float32),
            pltpu.VMEM((chunk,), jnp.float32),
            pltpu.VMEM((chunk,), jnp.float32),
            pltpu.VMEM((16,), jnp.float32),
            pltpu.VMEM((16,), jnp.float32),
            pltpu.VMEM((chunk,), jnp.float32),
            pltpu.VMEM((chunk,), jnp.float32),
            pltpu.VMEM_SHARED((QSP,), jnp.float32),
            pltpu.VMEM_SHARED((UPS,), jnp.int32),
            pltpu.VMEM_SHARED((128,), jnp.float32),
            pltpu.VMEM_SHARED((128,), jnp.float32),
            pltpu.SemaphoreType.DMA,
            pltpu.SemaphoreType.DMA,
            pltpu.SemaphoreType.DMA,
            pltpu.SemaphoreType.DMA,
        ],
    )
    def sc_combine(qids_hbm, uids_hbm, upair_hbm, vids_hbm, clicks_hbm,
                   q_proj_hbm, u_pairs_hbm, ve_hbm, vr_hbm,
                   pe_hbm, pr_hbm, dce_hbm, dcr_hbm,
                   pred_hbm, rels_hbm,
                   qi_v, ui_v, upi_v, vi_v, ci_v, qv_v, uv_v, vve_v, vvr_v,
                   pe_v, pr_v, dce_v, dcr_v, op_v, or_v,
                   q_sh, u_sh, ve_sh, vr_sh,
                   sem0, sem1, sem2, sem3):
        sid = lax.axis_index("s")
        wid = sid * nc + lax.axis_index("c")
        base = wid * chunk
        sl_all = pl.ds(base, chunk)
        upart = UPS // ns
        u0 = sid * upart
        pltpu.sync_copy(u_pairs_hbm.at[pl.ds(u0, upart)],
                        u_sh.at[pl.ds(u0, upart)])
        qpart = QSP // ns
        q0 = sid * qpart
        pltpu.sync_copy(q_proj_hbm.at[pl.ds(q0, qpart)],
                        q_sh.at[pl.ds(q0, qpart)])

        @pl.when(sid == 0)
        def _stage_small():
            pltpu.sync_copy(ve_hbm, ve_sh)
            pltpu.sync_copy(vr_hbm, vr_sh)

        pltpu.sync_copy(qids_hbm.at[sl_all], qi_v)
        pltpu.sync_copy(uids_hbm.at[sl_all], ui_v)
        pltpu.sync_copy(upair_hbm.at[sl_all], upi_v)
        pltpu.sync_copy(vids_hbm.at[sl_all], vi_v)
        pltpu.sync_copy(clicks_hbm.at[sl_all], ci_v)
        pltpu.sync_copy(pe_hbm, pe_v)
        pltpu.sync_copy(pr_hbm, pr_v)
        pltpu.sync_copy(dce_hbm, dce_v)
        pltpu.sync_copy(dcr_hbm, dcr_v)
        plsc.subcore_barrier()
        copies = []
        cf_ = chunk // NCH_F
        for c in range(NCH_F):
            sl = pl.ds(c * cf_, cf_)
            copies.append(pltpu.async_copy(u_sh.at[upi_v.at[sl]],
                                           uv_v.at[sl], sem1))
            copies.append(pltpu.async_copy(q_sh.at[qi_v.at[sl]],
                                           qv_v.at[sl], sem0))
            copies.append(pltpu.async_copy(ve_sh.at[vi_v.at[sl]],
                                           vve_v.at[sl], sem2))
            copies.append(pltpu.async_copy(vr_sh.at[vi_v.at[sl]],
                                           vvr_v.at[sl], sem3))
        for cp in copies:
            cp.wait()

        dce = dce_v[...]
        dcr = dcr_v[...]
        one = jnp.full((16,), 1, jnp.int32)
        himask = jnp.full((16,), -65536, jnp.int32)
        sixteen = jnp.full((16,), 16, jnp.int32)

        def body(g, carry):
            sl = pl.ds(g * 16, 16)
            up = uv_v[sl]
            lsb = lax.bitwise_and(ui_v[sl], one)
            sham = lax.mul(lax.sub(one, lsb), sixteen)
            ubits = lax.bitwise_and(lax.shift_left(up, sham), himask)
            uval = lax.bitcast_convert_type(ubits, jnp.float32)
            cf = ci_v[sl].astype(jnp.float32)
            ex_logit = vve_v[sl] + cf * dce + pe_v[sl]
            rl_logit = qv_v[sl] + uval + vvr_v[sl] + cf * dcr + pr_v[sl]
            ex = 1.0 / (1.0 + jnp.exp(-ex_logit))
            rl = 1.0 / (1.0 + jnp.exp(-rl_logit))
            or_v[sl] = rl
            op_v[sl] = rl * ex
            return carry

        lax.fori_loop(0, groups, body, 0)

        pltpu.sync_copy(op_v, pred_hbm.at[sl_all])
        pltpu.sync_copy(or_v, rels_hbm.at[sl_all])

    return sc_combine


_sc_combine = None


def kernel(qids, uids, vids, clicks, q_table, u_table, v_table, click_table,
           pos_table, w_exam, b_exam, w_rel, b_rel):
    global _sc_combine
    if _sc_combine is None:
        _sc_combine = _make_sc_combine()

    u_proj = _project(u_table.T, w_rel[D:2 * D].reshape(D, 1), UBLK,
                      jnp.bfloat16)
    q_proj = _project(q_table.T, w_rel[:D].reshape(D, 1), QBLK)

    zpad = jnp.zeros((SPAD - SMALL, D), jnp.float32)
    small_table = jnp.concatenate([v_table, click_table, pos_table, zpad],
                                  axis=0)
    w_e_mat = jnp.concatenate([
        jnp.broadcast_to(w_exam[:D], (VS, D)),
        jnp.broadcast_to(w_exam[D:2 * D], (CS, D)),
        jnp.broadcast_to(w_exam[2 * D:], (S, D)),
        zpad,
    ], axis=0)
    w_r_mat = jnp.concatenate([
        jnp.broadcast_to(w_rel[2 * D:3 * D], (VS, D)),
        jnp.broadcast_to(w_rel[3 * D:4 * D], (CS, D)),
        jnp.broadcast_to(w_rel[4 * D:], (S, D)),
        zpad,
    ], axis=0)
    rows = jnp.arange(SPAD)
    click_row = ((rows >= VS) & (rows < VS + CS)).astype(jnp.float32)
    bias_e = (click_row * b_exam[0]).reshape(SPAD, 1)
    bias_r = (click_row * b_rel[0]).reshape(SPAD, 1)
    small_e, small_r = _project_small(small_table, w_e_mat, w_r_mat,
                                      bias_e, bias_r)

    v_e = jnp.concatenate([small_e[:VS], jnp.zeros((128 - VS,), jnp.float32)])
    v_r = jnp.concatenate([small_r[:VS], jnp.zeros((128 - VS,), jnp.float32)])
    ce0, ce1 = small_e[VS], small_e[VS + 1]
    cr0, cr1 = small_r[VS], small_r[VS + 1]
    p_e = small_e[VS + CS:SMALL]
    p_r = small_r[VS + CS:SMALL]
    pe_chunk = jnp.tile(p_e, 128) + ce0
    pr_chunk = jnp.tile(p_r, 128) + cr0
    dce16 = jnp.broadcast_to(ce1 - ce0, (16,))
    dcr16 = jnp.broadcast_to(cr1 - cr0, (16,))

    u_pairs = lax.bitcast_convert_type(u_proj.reshape(UPS, 2), jnp.int32)
    uids_f = uids.reshape(BS)
    pred_f, rels_f = _sc_combine(
        qids.reshape(BS), uids_f, lax.shift_right_logical(uids_f, 1),
        vids.reshape(BS), clicks.reshape(BS), q_proj, u_pairs, v_e, v_r,
        pe_chunk, pr_chunk, dce16, dcr16)
    return pred_f.reshape(B, S), rels_f.reshape(B, S)


# speedup vs baseline: 58.2322x; 3.5945x over previous
import functools

import jax
import jax.numpy as jnp
from jax import lax
from jax.experimental import pallas as pl
from jax.experimental.pallas import tpu as pltpu
from jax.experimental.pallas import tpu_sc as plsc

B, S, D = 4096, 50, 32
QS, US, VS, CS = 100000, 1000000, 100, 2
SMALL = VS + CS + S
SPAD = 256
BS = B * S
UBLK, QBLK = 16384, 4096
USP = ((US + 2 * UBLK - 1) // (2 * UBLK)) * 2 * UBLK
QSP = ((QS + QBLK - 1) // QBLK) * QBLK
UPS = USP // 2
NCH_F = 10


def _proj_body_f32(x_ref, w_ref, o_ref):
    o_ref[...] = jnp.sum(x_ref[...] * w_ref[...], axis=0)


def _rne_bf16_bits(x):
    r = lax.bitcast_convert_type(x, jnp.int32)
    return lax.shift_right_logical(
        r + 0x7FFF + lax.bitwise_and(lax.shift_right_logical(r, 16), 1), 16)


def _pack_body(xa_ref, xb_ref, xq_ref, wu_ref, wq_ref, o_ref, oq_ref):
    a = jnp.sum(xa_ref[...] * wu_ref[...], axis=0)
    b = jnp.sum(xb_ref[...] * wu_ref[...], axis=0)
    o_ref[...] = lax.bitwise_or(_rne_bf16_bits(a),
                                lax.shift_left(_rne_bf16_bits(b), 16))
    oq_ref[...] = jnp.sum(xq_ref[...] * wq_ref[...], axis=0)


def _project_pack(table_t, q_t, wu_col, wq_col, blk, nblk, qblk, qnblk):
    qlast = qnblk - 1
    out, outq = pl.pallas_call(
        _pack_body,
        grid=(nblk,),
        in_specs=[
            pl.BlockSpec((D, blk), lambda i: (0, i)),
            pl.BlockSpec((D, blk), lambda i: (0, i + nblk)),
            pl.BlockSpec((D, qblk), lambda i: (0, jnp.minimum(i, qlast))),
            pl.BlockSpec((D, 1), lambda i: (0, 0)),
            pl.BlockSpec((D, 1), lambda i: (0, 0)),
        ],
        out_specs=(pl.BlockSpec((blk,), lambda i: (i,)),
                   pl.BlockSpec((qblk,), lambda i: (jnp.minimum(i, qlast),))),
        out_shape=(jax.ShapeDtypeStruct((nblk * blk,), jnp.int32),
                   jax.ShapeDtypeStruct((qnblk * qblk,), jnp.float32)),
    )(table_t, table_t, q_t, wu_col, wq_col)
    return out, outq


def _project(table_t, w_col, blk, dtype=jnp.float32):
    n = table_t.shape[1]
    grid = (n + blk - 1) // blk
    body = _proj_body_f32
    out = pl.pallas_call(
        body,
        grid=(grid,),
        in_specs=[
            pl.BlockSpec((D, blk), lambda i: (0, i)),
            pl.BlockSpec((D, 1), lambda i: (0, 0)),
        ],
        out_specs=pl.BlockSpec((blk,), lambda i: (i,)),
        out_shape=jax.ShapeDtypeStruct((grid * blk,), dtype),
    )(table_t, w_col)
    return out


def _small_body(x_ref, we_ref, wr_ref, be_ref, br_ref, oe_ref, or_ref,
                ov_ref):
    x = x_ref[...]
    oe = jnp.sum(x * we_ref[...], axis=1, keepdims=True) + be_ref[...]
    orr = jnp.sum(x * wr_ref[...], axis=1, keepdims=True) + br_ref[...]
    oe_ref[...] = oe
    or_ref[...] = orr
    ov_ref[...] = lax.bitwise_or(_rne_bf16_bits(oe),
                                 lax.shift_left(_rne_bf16_bits(orr), 16))


def _project_small(small_table, w_e_mat, w_r_mat, bias_e, bias_r):
    oe, orr, ov = pl.pallas_call(
        _small_body,
        out_shape=(
            jax.ShapeDtypeStruct((SPAD, 1), jnp.float32),
            jax.ShapeDtypeStruct((SPAD, 1), jnp.float32),
            jax.ShapeDtypeStruct((SPAD, 1), jnp.int32),
        ),
    )(small_table, w_e_mat, w_r_mat, bias_e, bias_r)
    return oe[:, 0], orr[:, 0], ov[:, 0]


def _make_sc_exam():
    info = plsc.get_sparse_core_info()
    nc, ns = info.num_cores, info.num_subcores
    nw = nc * ns
    chunk = BS // nw
    groups = chunk // 16

    mesh = plsc.VectorSubcoreMesh(core_axis_name="c", subcore_axis_name="s")

    @functools.partial(
        pl.kernel,
        mesh=mesh,
        out_type=(
            jax.ShapeDtypeStruct((BS,), jnp.float32),
            jax.ShapeDtypeStruct((BS,), jnp.float32),
        ),
        scratch_types=[
            pltpu.VMEM((chunk,), jnp.int32),
            pltpu.VMEM((chunk,), jnp.int32),
            pltpu.VMEM((chunk,), jnp.int32),
            pltpu.VMEM((chunk,), jnp.float32),
            pltpu.VMEM((chunk,), jnp.float32),
            pltpu.VMEM((16,), jnp.float32),
            pltpu.VMEM((16,), jnp.float32),
            pltpu.VMEM((chunk,), jnp.float32),
            pltpu.VMEM((chunk,), jnp.float32),
            pltpu.VMEM_SHARED((128,), jnp.int32),
            pltpu.SemaphoreType.DMA,
        ],
    )
    def sc_exam(vids_hbm, clicks_hbm, vpk_hbm, pe_hbm, pr_hbm,
                dce_hbm, dcr_hbm, ex_hbm, rp_hbm,
                vi_v, ci_v, vv_v, pe_v, pr_v, dce_v, dcr_v, oe_v, orp_v,
                v_sh, sem0):
        sid = lax.axis_index("s")
        wid = sid * nc + lax.axis_index("c")
        base = wid * chunk
        sl_all = pl.ds(base, chunk)

        @pl.when(sid == 0)
        def _stage_small():
            pltpu.sync_copy(vpk_hbm, v_sh)

        pltpu.sync_copy(vids_hbm.at[sl_all], vi_v)
        pltpu.sync_copy(clicks_hbm.at[sl_all], ci_v)
        pltpu.sync_copy(pe_hbm, pe_v)
        pltpu.sync_copy(pr_hbm, pr_v)
        pltpu.sync_copy(dce_hbm, dce_v)
        pltpu.sync_copy(dcr_hbm, dcr_v)
        plsc.subcore_barrier()
        copies = []
        cf_ = chunk // NCH_F
        for c in range(NCH_F):
            sl = pl.ds(c * cf_, cf_)
            copies.append(pltpu.async_copy(v_sh.at[vi_v.at[sl]],
                                           vv_v.at[sl], sem0))
        for cp in copies:
            cp.wait()

        dce = dce_v[...]
        dcr = dcr_v[...]
        himask = jnp.full((16,), -65536, jnp.int32)
        sixteen = jnp.full((16,), 16, jnp.int32)

        def body(g, carry):
            sl = pl.ds(g * 16, 16)
            vw = vv_v[sl]
            ve = lax.bitcast_convert_type(lax.shift_left(vw, sixteen),
                                          jnp.float32)
            vr = lax.bitcast_convert_type(lax.bitwise_and(vw, himask),
                                          jnp.float32)
            cf = ci_v[sl].astype(jnp.float32)
            ex = 1.0 / (1.0 + jnp.exp(-(ve + cf * dce + pe_v[sl])))
            oe_v[sl] = ex
            orp_v[sl] = vr + cf * dcr + pr_v[sl]
            return carry

        lax.fori_loop(0, groups, body, 0)

        pltpu.sync_copy(oe_v, ex_hbm.at[sl_all])
        pltpu.sync_copy(orp_v, rp_hbm.at[sl_all])

    return sc_exam


def _make_sc_rel():
    info = plsc.get_sparse_core_info()
    nc, ns = info.num_cores, info.num_subcores
    nw = nc * ns
    chunk = BS // nw
    groups = chunk // 16

    mesh = plsc.VectorSubcoreMesh(core_axis_name="c", subcore_axis_name="s")

    @functools.partial(
        pl.kernel,
        mesh=mesh,
        out_type=(
            jax.ShapeDtypeStruct((BS,), jnp.float32),
            jax.ShapeDtypeStruct((BS,), jnp.float32),
        ),
        scratch_types=[
            pltpu.VMEM((chunk,), jnp.int32),
            pltpu.VMEM((chunk,), jnp.int32),
            pltpu.VMEM((chunk,), jnp.int32),
            pltpu.VMEM((chunk,), jnp.float32),
            pltpu.VMEM((chunk,), jnp.int32),
            pltpu.VMEM((chunk,), jnp.float32),
            pltpu.VMEM((chunk,), jnp.float32),
            pltpu.VMEM((chunk,), jnp.float32),
            pltpu.VMEM((chunk,), jnp.float32),
            pltpu.VMEM_SHARED((QSP,), jnp.float32),
            pltpu.VMEM_SHARED((UPS,), jnp.int32),
            pltpu.SemaphoreType.DMA,
            pltpu.SemaphoreType.DMA,
        ],
    )
    def sc_rel(qids_hbm, uids_hbm, upair_hbm, q_proj_hbm, u_pairs_hbm,
               ex_hbm, rp_hbm, pred_hbm, rels_hbm,
               qi_v, ui_v, upi_v, qv_v, uv_v, ex_v, rp_v, op_v, or_v,
               q_sh, u_sh, sem0, sem1):
        sid = lax.axis_index("s")
        wid = sid * nc + lax.axis_index("c")
        base = wid * chunk
        sl_all = pl.ds(base, chunk)
        upart = UPS // ns
        u0 = sid * upart
        pltpu.sync_copy(u_pairs_hbm.at[pl.ds(u0, upart)],
                        u_sh.at[pl.ds(u0, upart)])
        qpart = QSP // ns
        q0 = sid * qpart
        pltpu.sync_copy(q_proj_hbm.at[pl.ds(q0, qpart)],
                        q_sh.at[pl.ds(q0, qpart)])
        pltpu.sync_copy(qids_hbm.at[sl_all], qi_v)
        pltpu.sync_copy(uids_hbm.at[sl_all], ui_v)
        pltpu.sync_copy(upair_hbm.at[sl_all], upi_v)
        pltpu.sync_copy(ex_hbm.at[sl_all], ex_v)
        pltpu.sync_copy(rp_hbm.at[sl_all], rp_v)
        plsc.subcore_barrier()
        copies = []
        cf_ = chunk // NCH_F
        for c in range(NCH_F):
            sl = pl.ds(c * cf_, cf_)
            copies.append(pltpu.async_copy(u_sh.at[upi_v.at[sl]],
                                           uv_v.at[sl], sem1))
            copies.append(pltpu.async_copy(q_sh.at[qi_v.at[sl]],
                                           qv_v.at[sl], sem0))
        for cp in copies:
            cp.wait()

        one = jnp.full((16,), 1, jnp.int32)
        zero = jnp.full((16,), 0, jnp.int32)
        upsv = jnp.full((16,), UPS, jnp.int32)
        himask = jnp.full((16,), -65536, jnp.int32)
        sixteen = jnp.full((16,), 16, jnp.int32)

        def body(g, carry):
            sl = pl.ds(g * 16, 16)
            up = uv_v[sl]
            hi = lax.select(ui_v[sl] >= upsv, one, zero)
            sham = lax.mul(lax.sub(one, hi), sixteen)
            ubits = lax.bitwise_and(lax.shift_left(up, sham), himask)
            uval = lax.bitcast_convert_type(ubits, jnp.float32)
            rl = 1.0 / (1.0 + jnp.exp(-(qv_v[sl] + uval + rp_v[sl])))
            or_v[sl] = rl
            op_v[sl] = rl * ex_v[sl]
            return carry

        lax.fori_loop(0, groups, body, 0)

        pltpu.sync_copy(op_v, pred_hbm.at[sl_all])
        pltpu.sync_copy(or_v, rels_hbm.at[sl_all])

    return sc_rel


_sc_exam = None
_sc_rel = None


def kernel(qids, uids, vids, clicks, q_table, u_table, v_table, click_table,
           pos_table, w_exam, b_exam, w_rel, b_rel):
    global _sc_exam, _sc_rel
    if _sc_exam is None:
        _sc_exam = _make_sc_exam()
        _sc_rel = _make_sc_rel()

    u_pairs, q_proj = _project_pack(
        u_table.T, q_table.T, w_rel[D:2 * D].reshape(D, 1),
        w_rel[:D].reshape(D, 1), UBLK, UPS // UBLK, QBLK, QSP // QBLK)

    zpad = jnp.zeros((SPAD - SMALL, D), jnp.float32)
    small_table = jnp.concatenate([v_table, click_table, pos_table, zpad],
                                  axis=0)
    w_e_mat = jnp.concatenate([
        jnp.broadcast_to(w_exam[:D], (VS, D)),
        jnp.broadcast_to(w_exam[D:2 * D], (CS, D)),
        jnp.broadcast_to(w_exam[2 * D:], (S, D)),
        zpad,
    ], axis=0)
    w_r_mat = jnp.concatenate([
        jnp.broadcast_to(w_rel[2 * D:3 * D], (VS, D)),
        jnp.broadcast_to(w_rel[3 * D:4 * D], (CS, D)),
        jnp.broadcast_to(w_rel[4 * D:], (S, D)),
        zpad,
    ], axis=0)
    rows = jnp.arange(SPAD)
    click_row = ((rows >= VS) & (rows < VS + CS)).astype(jnp.float32)
    bias_e = (click_row * b_exam[0]).reshape(SPAD, 1)
    bias_r = (click_row * b_rel[0]).reshape(SPAD, 1)
    small_e, small_r, small_v = _project_small(small_table, w_e_mat, w_r_mat,
                                               bias_e, bias_r)

    v_pk = jnp.concatenate([small_v[:VS], jnp.zeros((128 - VS,), jnp.int32)])
    ce0, ce1 = small_e[VS], small_e[VS + 1]
    cr0, cr1 = small_r[VS], small_r[VS + 1]
    p_e = small_e[VS + CS:SMALL]
    p_r = small_r[VS + CS:SMALL]
    pe_chunk = jnp.tile(p_e, 128) + ce0
    pr_chunk = jnp.tile(p_r, 128) + cr0
    dce16 = jnp.broadcast_to(ce1 - ce0, (16,))
    dcr16 = jnp.broadcast_to(cr1 - cr0, (16,))

    ex_f, rp_f = _sc_exam(vids.reshape(BS), clicks.reshape(BS), v_pk,
                          pe_chunk, pr_chunk, dce16, dcr16)

    uids_f = uids.reshape(BS)
    upair_idx = jnp.where(uids_f >= UPS, uids_f - UPS, uids_f)
    pred_f, rels_f = _sc_rel(
        qids.reshape(BS), uids_f, upair_idx, q_proj, u_pairs, ex_f, rp_f)
    return pred_f.reshape(B, S), rels_f.reshape(B, S)


# speedup vs baseline: 61.7211x; 1.0599x over previous
import functools

import jax
import jax.numpy as jnp
from jax import lax
from jax.experimental import pallas as pl
from jax.experimental.pallas import tpu as pltpu
from jax.experimental.pallas import tpu_sc as plsc

B, S, D = 4096, 50, 32
QS, US, VS, CS = 100000, 1000000, 100, 2
SMALL = VS + CS + S
SPAD = 256
BS = B * S
UBLK, QBLK = 31744, 4096
USP = ((US + 2 * UBLK - 1) // (2 * UBLK)) * 2 * UBLK
QSP = ((QS + QBLK - 1) // QBLK) * QBLK
UPS = USP // 2
NCH_F = 10


def _proj_body_f32(x_ref, w_ref, o_ref):
    o_ref[...] = jnp.sum(x_ref[...] * w_ref[...], axis=0)


def _rne_bf16_bits(x):
    r = lax.bitcast_convert_type(x, jnp.int32)
    return lax.shift_right_logical(
        r + 0x7FFF + lax.bitwise_and(lax.shift_right_logical(r, 16), 1), 16)


def _pack_body(xa_ref, xb_ref, xq_ref, wu_ref, wq_ref, o_ref, oq_ref):
    a = jnp.sum(xa_ref[...] * wu_ref[...], axis=0)
    b = jnp.sum(xb_ref[...] * wu_ref[...], axis=0)
    o_ref[...] = lax.bitwise_or(_rne_bf16_bits(a),
                                lax.shift_left(_rne_bf16_bits(b), 16))
    oq_ref[...] = jnp.sum(xq_ref[...] * wq_ref[...], axis=0)


def _project_pack(table_t, q_t, wu_col, wq_col, blk, nblk, qblk, qnblk):
    qlast = qnblk - 1
    out, outq = pl.pallas_call(
        _pack_body,
        grid=(nblk,),
        in_specs=[
            pl.BlockSpec((D, blk), lambda i: (0, i)),
            pl.BlockSpec((D, blk), lambda i: (0, i + nblk)),
            pl.BlockSpec((D, qblk), lambda i: (0, jnp.minimum(i, qlast))),
            pl.BlockSpec((D, 1), lambda i: (0, 0)),
            pl.BlockSpec((D, 1), lambda i: (0, 0)),
        ],
        out_specs=(pl.BlockSpec((blk,), lambda i: (i,)),
                   pl.BlockSpec((qblk,), lambda i: (jnp.minimum(i, qlast),))),
        out_shape=(jax.ShapeDtypeStruct((nblk * blk,), jnp.int32),
                   jax.ShapeDtypeStruct((qnblk * qblk,), jnp.float32)),
    )(table_t, table_t, q_t, wu_col, wq_col)
    return out, outq


def _project(table_t, w_col, blk, dtype=jnp.float32):
    n = table_t.shape[1]
    grid = (n + blk - 1) // blk
    body = _proj_body_f32
    out = pl.pallas_call(
        body,
        grid=(grid,),
        in_specs=[
            pl.BlockSpec((D, blk), lambda i: (0, i)),
            pl.BlockSpec((D, 1), lambda i: (0, 0)),
        ],
        out_specs=pl.BlockSpec((blk,), lambda i: (i,)),
        out_shape=jax.ShapeDtypeStruct((grid * blk,), dtype),
    )(table_t, w_col)
    return out


def _small_body(x_ref, we_ref, wr_ref, be_ref, br_ref, oe_ref, or_ref,
                ov_ref):
    x = x_ref[...]
    oe = jnp.sum(x * we_ref[...], axis=1, keepdims=True) + be_ref[...]
    orr = jnp.sum(x * wr_ref[...], axis=1, keepdims=True) + br_ref[...]
    oe_ref[...] = oe
    or_ref[...] = orr
    ov_ref[...] = lax.bitwise_or(_rne_bf16_bits(oe),
                                 lax.shift_left(_rne_bf16_bits(orr), 16))


def _project_small(small_table, w_e_mat, w_r_mat, bias_e, bias_r):
    oe, orr, ov = pl.pallas_call(
        _small_body,
        out_shape=(
            jax.ShapeDtypeStruct((SPAD, 1), jnp.float32),
            jax.ShapeDtypeStruct((SPAD, 1), jnp.float32),
            jax.ShapeDtypeStruct((SPAD, 1), jnp.int32),
        ),
    )(small_table, w_e_mat, w_r_mat, bias_e, bias_r)
    return oe[:, 0], orr[:, 0], ov[:, 0]


def _make_sc_exam():
    info = plsc.get_sparse_core_info()
    nc, ns = info.num_cores, info.num_subcores
    nw = nc * ns
    chunk = BS // nw
    groups = chunk // 16

    mesh = plsc.VectorSubcoreMesh(core_axis_name="c", subcore_axis_name="s")

    @functools.partial(
        pl.kernel,
        mesh=mesh,
        out_type=(
            jax.ShapeDtypeStruct((BS,), jnp.float32),
            jax.ShapeDtypeStruct((BS,), jnp.float32),
        ),
        scratch_types=[
            pltpu.VMEM((chunk,), jnp.int32),
            pltpu.VMEM((chunk,), jnp.int32),
            pltpu.VMEM((chunk,), jnp.int32),
            pltpu.VMEM((chunk,), jnp.float32),
            pltpu.VMEM((chunk,), jnp.float32),
            pltpu.VMEM((16,), jnp.float32),
            pltpu.VMEM((16,), jnp.float32),
            pltpu.VMEM((chunk,), jnp.float32),
            pltpu.VMEM((chunk,), jnp.float32),
            pltpu.VMEM_SHARED((128,), jnp.int32),
            pltpu.SemaphoreType.DMA,
        ],
    )
    def sc_exam(vids_hbm, clicks_hbm, vpk_hbm, pe_hbm, pr_hbm,
                dce_hbm, dcr_hbm, ex_hbm, rp_hbm,
                vi_v, ci_v, vv_v, pe_v, pr_v, dce_v, dcr_v, oe_v, orp_v,
                v_sh, sem0):
        sid = lax.axis_index("s")
        wid = sid * nc + lax.axis_index("c")
        base = wid * chunk
        sl_all = pl.ds(base, chunk)

        @pl.when(sid == 0)
        def _stage_small():
            pltpu.sync_copy(vpk_hbm, v_sh)

        pltpu.sync_copy(vids_hbm.at[sl_all], vi_v)
        pltpu.sync_copy(clicks_hbm.at[sl_all], ci_v)
        pltpu.sync_copy(pe_hbm, pe_v)
        pltpu.sync_copy(pr_hbm, pr_v)
        pltpu.sync_copy(dce_hbm, dce_v)
        pltpu.sync_copy(dcr_hbm, dcr_v)
        plsc.subcore_barrier()
        copies = []
        cf_ = chunk // NCH_F
        for c in range(NCH_F):
            sl = pl.ds(c * cf_, cf_)
            copies.append(pltpu.async_copy(v_sh.at[vi_v.at[sl]],
                                           vv_v.at[sl], sem0))
        for cp in copies:
            cp.wait()

        dce = dce_v[...]
        dcr = dcr_v[...]
        himask = jnp.full((16,), -65536, jnp.int32)
        sixteen = jnp.full((16,), 16, jnp.int32)

        def body(g, carry):
            sl = pl.ds(g * 16, 16)
            vw = vv_v[sl]
            ve = lax.bitcast_convert_type(lax.shift_left(vw, sixteen),
                                          jnp.float32)
            vr = lax.bitcast_convert_type(lax.bitwise_and(vw, himask),
                                          jnp.float32)
            cf = ci_v[sl].astype(jnp.float32)
            ex = 1.0 / (1.0 + jnp.exp(-(ve + cf * dce + pe_v[sl])))
            oe_v[sl] = ex
            orp_v[sl] = vr + cf * dcr + pr_v[sl]
            return carry

        lax.fori_loop(0, groups, body, 0)

        pltpu.sync_copy(oe_v, ex_hbm.at[sl_all])
        pltpu.sync_copy(orp_v, rp_hbm.at[sl_all])

    return sc_exam


def _make_sc_rel():
    info = plsc.get_sparse_core_info()
    nc, ns = info.num_cores, info.num_subcores
    nw = nc * ns
    chunk = BS // nw
    groups = chunk // 16

    mesh = plsc.VectorSubcoreMesh(core_axis_name="c", subcore_axis_name="s")

    @functools.partial(
        pl.kernel,
        mesh=mesh,
        out_type=(
            jax.ShapeDtypeStruct((BS,), jnp.float32),
            jax.ShapeDtypeStruct((BS,), jnp.float32),
        ),
        scratch_types=[
            pltpu.VMEM((chunk,), jnp.int32),
            pltpu.VMEM((chunk,), jnp.int32),
            pltpu.VMEM((chunk,), jnp.int32),
            pltpu.VMEM((chunk,), jnp.float32),
            pltpu.VMEM((chunk,), jnp.int32),
            pltpu.VMEM((chunk,), jnp.float32),
            pltpu.VMEM((chunk,), jnp.float32),
            pltpu.VMEM((chunk,), jnp.float32),
            pltpu.VMEM((chunk,), jnp.float32),
            pltpu.VMEM_SHARED((QSP,), jnp.float32),
            pltpu.VMEM_SHARED((UPS,), jnp.int32),
            pltpu.SemaphoreType.DMA,
            pltpu.SemaphoreType.DMA,
        ],
    )
    def sc_rel(qids_hbm, uids_hbm, upair_hbm, q_proj_hbm, u_pairs_hbm,
               ex_hbm, rp_hbm, pred_hbm, rels_hbm,
               qi_v, ui_v, upi_v, qv_v, uv_v, ex_v, rp_v, op_v, or_v,
               q_sh, u_sh, sem0, sem1):
        sid = lax.axis_index("s")
        wid = sid * nc + lax.axis_index("c")
        base = wid * chunk
        sl_all = pl.ds(base, chunk)
        upart = UPS // ns
        u0 = sid * upart
        pltpu.sync_copy(u_pairs_hbm.at[pl.ds(u0, upart)],
                        u_sh.at[pl.ds(u0, upart)])
        qpart = QSP // ns
        q0 = sid * qpart
        pltpu.sync_copy(q_proj_hbm.at[pl.ds(q0, qpart)],
                        q_sh.at[pl.ds(q0, qpart)])
        pltpu.sync_copy(qids_hbm.at[sl_all], qi_v)
        pltpu.sync_copy(uids_hbm.at[sl_all], ui_v)
        pltpu.sync_copy(upair_hbm.at[sl_all], upi_v)
        pltpu.sync_copy(ex_hbm.at[sl_all], ex_v)
        pltpu.sync_copy(rp_hbm.at[sl_all], rp_v)
        plsc.subcore_barrier()
        copies = []
        cf_ = chunk // NCH_F
        for c in range(NCH_F):
            sl = pl.ds(c * cf_, cf_)
            copies.append(pltpu.async_copy(u_sh.at[upi_v.at[sl]],
                                           uv_v.at[sl], sem1))
            copies.append(pltpu.async_copy(q_sh.at[qi_v.at[sl]],
                                           qv_v.at[sl], sem0))
        for cp in copies:
            cp.wait()

        one = jnp.full((16,), 1, jnp.int32)
        zero = jnp.full((16,), 0, jnp.int32)
        upsv = jnp.full((16,), UPS, jnp.int32)
        himask = jnp.full((16,), -65536, jnp.int32)
        sixteen = jnp.full((16,), 16, jnp.int32)

        def body(g, carry):
            sl = pl.ds(g * 16, 16)
            up = uv_v[sl]
            hi = lax.select(ui_v[sl] >= upsv, one, zero)
            sham = lax.mul(lax.sub(one, hi), sixteen)
            ubits = lax.bitwise_and(lax.shift_left(up, sham), himask)
            uval = lax.bitcast_convert_type(ubits, jnp.float32)
            rl = 1.0 / (1.0 + jnp.exp(-(qv_v[sl] + uval + rp_v[sl])))
            or_v[sl] = rl
            op_v[sl] = rl * ex_v[sl]
            return carry

        lax.fori_loop(0, groups, body, 0)

        pltpu.sync_copy(op_v, pred_hbm.at[sl_all])
        pltpu.sync_copy(or_v, rels_hbm.at[sl_all])

    return sc_rel


_sc_exam = None
_sc_rel = None


def kernel(qids, uids, vids, clicks, q_table, u_table, v_table, click_table,
           pos_table, w_exam, b_exam, w_rel, b_rel):
    global _sc_exam, _sc_rel
    if _sc_exam is None:
        _sc_exam = _make_sc_exam()
        _sc_rel = _make_sc_rel()

    u_pairs, q_proj = _project_pack(
        u_table.T, q_table.T, w_rel[D:2 * D].reshape(D, 1),
        w_rel[:D].reshape(D, 1), UBLK, UPS // UBLK, QBLK, QSP // QBLK)

    zpad = jnp.zeros((SPAD - SMALL, D), jnp.float32)
    small_table = jnp.concatenate([v_table, click_table, pos_table, zpad],
                                  axis=0)
    w_e_mat = jnp.concatenate([
        jnp.broadcast_to(w_exam[:D], (VS, D)),
        jnp.broadcast_to(w_exam[D:2 * D], (CS, D)),
        jnp.broadcast_to(w_exam[2 * D:], (S, D)),
        zpad,
    ], axis=0)
    w_r_mat = jnp.concatenate([
        jnp.broadcast_to(w_rel[2 * D:3 * D], (VS, D)),
        jnp.broadcast_to(w_rel[3 * D:4 * D], (CS, D)),
        jnp.broadcast_to(w_rel[4 * D:], (S, D)),
        zpad,
    ], axis=0)
    rows = jnp.arange(SPAD)
    click_row = ((rows >= VS) & (rows < VS + CS)).astype(jnp.float32)
    bias_e = (click_row * b_exam[0]).reshape(SPAD, 1)
    bias_r = (click_row * b_rel[0]).reshape(SPAD, 1)
    small_e, small_r, small_v = _project_small(small_table, w_e_mat, w_r_mat,
                                               bias_e, bias_r)

    v_pk = jnp.concatenate([small_v[:VS], jnp.zeros((128 - VS,), jnp.int32)])
    ce0, ce1 = small_e[VS], small_e[VS + 1]
    cr0, cr1 = small_r[VS], small_r[VS + 1]
    p_e = small_e[VS + CS:SMALL]
    p_r = small_r[VS + CS:SMALL]
    pe_chunk = jnp.tile(p_e, 128) + ce0
    pr_chunk = jnp.tile(p_r, 128) + cr0
    dce16 = jnp.broadcast_to(ce1 - ce0, (16,))
    dcr16 = jnp.broadcast_to(cr1 - cr0, (16,))

    ex_f, rp_f = _sc_exam(vids.reshape(BS), clicks.reshape(BS), v_pk,
                          pe_chunk, pr_chunk, dce16, dcr16)

    uids_f = uids.reshape(BS)
    upair_idx = jnp.where(uids_f >= UPS, uids_f - UPS, uids_f)
    pred_f, rels_f = _sc_rel(
        qids.reshape(BS), uids_f, upair_idx, q_proj, u_pairs, ex_f, rp_f)
    return pred_f.reshape(B, S), rels_f.reshape(B, S)
